# baseline (speedup 1.0000x reference)
import functools
import numpy as np
import jax
import jax.numpy as jnp
from jax import lax
from jax.experimental import pallas as pl
from jax.experimental.pallas import tpu as pltpu
from jax.experimental.pallas import tpu_sc as plsc

_B, _L, _V, _D = 1024, 200, 100000, 64

_NC, _NS = 2, 16
_W = _NC * _NS
_G = 128
_NBT = _B // _G
_NLW = _L // (_W // _NBT)
_NDT = _D // 8


def _pos_encoding_host():
    pos = np.arange(_L)[:, np.newaxis]
    i = np.arange(_D)[np.newaxis, :]
    angle_rates = 1.0 / np.power(10000, 2 * (i // 2) / np.float32(_D))
    angles = pos * angle_rates
    angles[:, 0::2] = np.sin(angles[:, 0::2])
    angles[:, 1::2] = np.cos(angles[:, 1::2])
    return np.asarray(angles, dtype=np.float32)


_MESH = plsc.VectorSubcoreMesh(core_axis_name="c", subcore_axis_name="s")


def _make_embed_pe_kernel(name):
    @functools.partial(
        pl.kernel,
        mesh=_MESH,
        out_type=jax.ShapeDtypeStruct((_L, _NDT, _NBT, 8, _G), jnp.float32),
        scratch_types=[
            pltpu.VMEM((7, 1, 8, _G), jnp.int32),
            pltpu.VMEM((2, _G, _D), jnp.float32),
            pltpu.VMEM((2, _D, _G + 1), jnp.float32),
            pltpu.VMEM((_NLW, _D), jnp.float32),
            pltpu.SemaphoreType.DMA((2,)),
            pltpu.SemaphoreType.DMA((2,)),
        ],
        compiler_params=pltpu.CompilerParams(
            use_tc_tiling_on_sc=False, needs_layout_passes=False),
        name=name,
    )
    def _embed_pe_kernel(pe_hbm, idxt_hbm, tab_hbm, out_hbm,
                         idx_v, gath_v, tr_v, pe_v, gsem, osem):
        wid = lax.axis_index("s") * _NC + lax.axis_index("c")
        bt = wid // (_W // _NBT)
        l0 = _NLW * lax.rem(wid, _W // _NBT)
        lt0 = l0 // 8
        pltpu.sync_copy(
            idxt_hbm.at[pl.ds(lt0, 7), pl.ds(bt, 1)], idx_v)
        pltpu.sync_copy(pe_hbm.at[pl.ds(l0, _NLW)], pe_v)
        lane16 = jax.lax.iota(jnp.int32, 16)
        drows = [lane16 + 16 * q for q in range(_D // 16)]

        def fire(u, b):
            rl = l0 + u - lt0 * 8
            pltpu.async_copy(
                tab_hbm.at[idx_v.at[rl // 8, 0, lax.rem(rl, 8)]],
                gath_v.at[b], gsem.at[b])

        def wait_gather(b):
            pltpu.make_async_copy(
                tab_hbm.at[pl.ds(0, _G)], gath_v.at[b], gsem.at[b]).wait()

        def wait_stores(b):
            pltpu.make_async_copy(
                tab_hbm.at[pl.ds(0, _G)], gath_v.at[b], osem.at[b]).wait()

        fire(0, 0)

        def unit_iter(u, carry):
            b = lax.rem(u, 2)

            @pl.when(u + 1 < _NLW)
            def _():
                fire(u + 1, 1 - b)

            wait_gather(b)

            @pl.when(u >= 2)
            def _():
                wait_stores(b)

            pe_q = [pe_v[u, pl.ds(16 * q, 16)] for q in range(_D // 16)]

            @plsc.parallel_loop(0, _G, unroll=8)
            def tr_body(r, b=b, pe_q=pe_q):
                rcol = lane16 * 0 + r
                for q in range(_D // 16):
                    val = gath_v[b, r, pl.ds(16 * q, 16)] + pe_q[q]
                    plsc.store_scatter(tr_v.at[b], [drows[q], rcol], val)

            for dt in range(_NDT):
                pltpu.async_copy(
                    tr_v.at[b, pl.ds(dt * 8, 8), pl.ds(0, _G)],
                    out_hbm.at[l0 + u, dt, bt],
                    osem.at[b],
                )
            return carry

        lax.fori_loop(0, _NLW, unit_iter, 0)
        wait_stores((_NLW - 1) % 2)
        wait_stores((_NLW - 2) % 2)

    return _embed_pe_kernel


_embed_g = _make_embed_pe_kernel("embed_pe_g")
_embed_e = _make_embed_pe_kernel("embed_pe_e")


def _assemble(out5):
    return out5.transpose(2, 4, 0, 1, 3).reshape(_B, _L, _D)


def _idx_tiles(text):
    return text.T.reshape(_L // 8, 8, _NBT, _G).transpose(0, 2, 1, 3)


def kernel(g_text, e_text, g_table, e_table):
    pe = jnp.asarray(_pos_encoding_host())
    gtp = jnp.pad(g_table, ((0, 0), (0, _D))).reshape(2 * _V, _D)
    etp = jnp.pad(e_table, ((0, 0), (0, _D))).reshape(2 * _V, _D)
    g_out = _embed_g(pe, _idx_tiles(g_text * 2), gtp)
    e_out = _embed_e(pe, _idx_tiles(e_text * 2), etp)
    return (_assemble(g_out), _assemble(e_out))


# speedup vs baseline: 1.0326x; 1.0326x over previous
import functools
import numpy as np
import jax
import jax.numpy as jnp
from jax import lax
from jax.experimental import pallas as pl
from jax.experimental.pallas import tpu as pltpu
from jax.experimental.pallas import tpu_sc as plsc

_B, _L, _V, _D = 1024, 200, 100000, 64

_NC, _NS = 2, 16
_W = _NC * _NS
_G = 128
_NBT = _B // _G
_NLW = _L // (_W // _NBT)
_NDT = _D // 8


def _pos_encoding_host():
    pos = np.arange(_L)[:, np.newaxis]
    i = np.arange(_D)[np.newaxis, :]
    angle_rates = 1.0 / np.power(10000, 2 * (i // 2) / np.float32(_D))
    angles = pos * angle_rates
    angles[:, 0::2] = np.sin(angles[:, 0::2])
    angles[:, 1::2] = np.cos(angles[:, 1::2])
    return np.asarray(angles, dtype=np.float32)


_MESH = plsc.VectorSubcoreMesh(core_axis_name="c", subcore_axis_name="s")


def _make_embed_pe_kernel(name):
    @functools.partial(
        pl.kernel,
        mesh=_MESH,
        out_type=jax.ShapeDtypeStruct((_L, _NDT, _NBT, 8, _G), jnp.float32),
        scratch_types=[
            pltpu.VMEM((_NLW, _G), jnp.int32),
            pltpu.VMEM((2, _G, _D), jnp.float32),
            pltpu.VMEM((2, _D, _G + 1), jnp.float32),
            pltpu.VMEM((_NLW, _D), jnp.float32),
            pltpu.SemaphoreType.DMA((2,)),
            pltpu.SemaphoreType.DMA((2,)),
        ],
        compiler_params=pltpu.CompilerParams(
            use_tc_tiling_on_sc=False, needs_layout_passes=False),
        name=name,
    )
    def _embed_pe_kernel(pe_hbm, idxt_hbm, tab_hbm, out_hbm,
                         idx_v, gath_v, tr_v, pe_v, gsem, osem):
        wid = lax.axis_index("s") * _NC + lax.axis_index("c")
        bt = wid // (_W // _NBT)
        l0 = _NLW * lax.rem(wid, _W // _NBT)
        pltpu.sync_copy(
            idxt_hbm.at[pl.ds(l0, _NLW), pl.ds(bt * _G, _G)], idx_v)
        pltpu.sync_copy(pe_hbm.at[pl.ds(l0, _NLW)], pe_v)
        lane16 = jax.lax.iota(jnp.int32, 16)
        drows = [lane16 + 16 * q for q in range(_D // 16)]

        def fire(u, b):
            pltpu.async_copy(
                tab_hbm.at[idx_v.at[u]], gath_v.at[b], gsem.at[b])

        def wait_gather(b):
            pltpu.make_async_copy(
                tab_hbm.at[pl.ds(0, _G)], gath_v.at[b], gsem.at[b]).wait()

        def wait_stores(b):
            pltpu.make_async_copy(
                tab_hbm.at[pl.ds(0, _G)], gath_v.at[b], osem.at[b]).wait()

        fire(0, 0)

        def unit_iter(u, carry):
            b = lax.rem(u, 2)

            @pl.when(u + 1 < _NLW)
            def _():
                fire(u + 1, 1 - b)

            wait_gather(b)

            @pl.when(u >= 2)
            def _():
                wait_stores(b)

            pe_q = [pe_v[u, pl.ds(16 * q, 16)] for q in range(_D // 16)]

            @plsc.parallel_loop(0, _G, unroll=8)
            def tr_body(r, b=b, pe_q=pe_q):
                rcol = lane16 * 0 + r
                for q in range(_D // 16):
                    val = gath_v[b, r, pl.ds(16 * q, 16)] + pe_q[q]
                    plsc.store_scatter(tr_v.at[b], [drows[q], rcol], val)

            for dt in range(_NDT):
                pltpu.async_copy(
                    tr_v.at[b, pl.ds(dt * 8, 8), pl.ds(0, _G)],
                    out_hbm.at[l0 + u, dt, bt],
                    osem.at[b],
                )
            return carry

        lax.fori_loop(0, _NLW, unit_iter, 0)
        wait_stores((_NLW - 1) % 2)
        wait_stores((_NLW - 2) % 2)

    return _embed_pe_kernel


_embed_g = _make_embed_pe_kernel("embed_pe_g")
_embed_e = _make_embed_pe_kernel("embed_pe_e")


def _assemble(out5):
    return out5.transpose(2, 4, 0, 1, 3).reshape(_B, _L, _D)


def kernel(g_text, e_text, g_table, e_table):
    pe = jnp.asarray(_pos_encoding_host())
    g_out = _embed_g(pe, g_text.T, g_table)
    e_out = _embed_e(pe, e_text.T, e_table)
    return (_assemble(g_out), _assemble(e_out))
